# Initial kernel scaffold; baseline (speedup 1.0000x reference)
import jax
import jax.numpy as jnp
from jax.experimental import pallas as pl


def kernel(x, edge_index, batch, W_emb, b_emb, Wq0, bq0, Wk0, bk0, Wv0, bv0, Ws0, bs0, Wq1, bq1, Wk1, bk1, Wv1, bv1, Ws1, bs1, W_fc, b_fc):
    raise NotImplementedError("write your pallas kernel here")


# speedup vs baseline: 5.4537x; 5.4537x over previous
import functools

import jax
import jax.numpy as jnp
from jax import lax
from jax.experimental import pallas as pl
from jax.experimental.pallas import tpu as pltpu
from jax.experimental.pallas import tpu_sc as plsc

N = 10000
E = 160000
D = 256
HID = 256
H = 4
C = 64
G = 16

NC = 2
NS = 16
L = 16
NW = NC * NS
NPAD = 10240
EPAD = 160256
ROWS_W = NPAD // NS
EA_PER_W = EPAD // NW
CHUNKS_A = EA_PER_W // L
EB_PER_W = EPAD // NS
CHUNKS_B = EB_PER_W // L

f32 = jnp.float32
i32 = jnp.int32


def _mm(x, w, b, bm=1280):
    n, k = x.shape
    m = w.shape[1]

    def body(x_ref, w_ref, b_ref, o_ref):
        o_ref[...] = jnp.dot(x_ref[...], w_ref[...],
                             preferred_element_type=f32) + b_ref[...]

    return pl.pallas_call(
        body,
        grid=(n // bm,),
        in_specs=[pl.BlockSpec((bm, k), lambda i: (i, 0)),
                  pl.BlockSpec((k, m), lambda i: (0, 0)),
                  pl.BlockSpec((1, m), lambda i: (0, 0))],
        out_specs=pl.BlockSpec((bm, m), lambda i: (i, 0)),
        out_shape=jax.ShapeDtypeStruct((n, m), f32),
    )(x, w, b)


def _rden(den):
    def body(a_ref, b_ref, o_ref):
        o_ref[...] = 1.0 / (a_ref[...] + b_ref[...] + 1e-16)

    return pl.pallas_call(
        body,
        grid=(8,),
        in_specs=[pl.BlockSpec((1280, 16), lambda i: (i, 0)),
                  pl.BlockSpec((1280, 16), lambda i: (i + 8, 0))],
        out_specs=pl.BlockSpec((1280, 16), lambda i: (i, 0)),
        out_shape=jax.ShapeDtypeStruct((NPAD, 16), f32),
    )(den, den)


def _reluadd(outflat, sproj):
    def body(l_ref, r_ref, s_ref, o_ref):
        cat = jnp.concatenate([l_ref[...], r_ref[...]], axis=1)
        o_ref[...] = jnp.maximum(cat + s_ref[...], 0.0)

    return pl.pallas_call(
        body,
        grid=(8,),
        in_specs=[pl.BlockSpec((1280, 128), lambda i: (i, 0)),
                  pl.BlockSpec((1280, 128), lambda i: (i + 8, 0)),
                  pl.BlockSpec((1280, 256), lambda i: (i, 0))],
        out_specs=pl.BlockSpec((1280, 256), lambda i: (i, 0)),
        out_shape=jax.ShapeDtypeStruct((NPAD, 256), f32),
    )(outflat, outflat, sproj)


def _pool(h, bat3, w_fc, b_fc):
    def body(h_ref, b_ref, w_ref, bias_ref, o_ref, pacc, cacc):
        i = pl.program_id(0)

        @pl.when(i == 0)
        def _():
            pacc[...] = jnp.zeros_like(pacc)
            cacc[...] = jnp.zeros_like(cacc)

        bat = b_ref[0]
        gids = lax.broadcasted_iota(i32, (G, 1000), 0)
        oh = jnp.where(bat == gids, 1.0, 0.0)
        pacc[...] += jnp.dot(oh, h_ref[...], preferred_element_type=f32)
        cacc[...] += jnp.dot(oh, jnp.ones((1000, 128), f32),
                             preferred_element_type=f32)

        @pl.when(i == 9)
        def _():
            cnt = jnp.maximum(cacc[:, :1], 1.0)
            pooled = pacc[...] / cnt
            o_ref[...] = jnp.dot(pooled, w_ref[...],
                                 preferred_element_type=f32) + bias_ref[...]

    return pl.pallas_call(
        body,
        grid=(10,),
        in_specs=[pl.BlockSpec((1000, 256), lambda i: (i, 0)),
                  pl.BlockSpec((1, 1, 1000), lambda i: (i, 0, 0)),
                  pl.BlockSpec((256, 1), lambda i: (0, 0)),
                  pl.BlockSpec((1, 1), lambda i: (0, 0))],
        out_specs=pl.BlockSpec((G, 1), lambda i: (0, 0)),
        out_shape=jax.ShapeDtypeStruct((G, 1), f32),
        scratch_shapes=[pltpu.VMEM((G, 256), f32), pltpu.VMEM((G, 128), f32)],
    )(h, bat3, w_fc, b_fc)


def _sc_phase_a(q, k, srcp, dstp):
    mesh = plsc.VectorSubcoreMesh(core_axis_name="c", subcore_axis_name="s")

    @functools.partial(
        pl.kernel,
        mesh=mesh,
        out_type=(jax.ShapeDtypeStruct((EPAD, 16), f32),
                  jax.ShapeDtypeStruct((2 * NPAD, 16), f32)),
        compiler_params=pltpu.CompilerParams(
            use_tc_tiling_on_sc=False, needs_layout_passes=False),
        scratch_types=[
            pltpu.VMEM_SHARED((NPAD, 16), f32),
            pltpu.VMEM((16,), i32),
            pltpu.VMEM((16,), i32),
            pltpu.VMEM((16, 256), f32),
            pltpu.VMEM((16, 256), f32),
            pltpu.VMEM((16, 16), f32),
            pltpu.SemaphoreType.DMA,
            pltpu.SemaphoreType.DMA,
        ],
    )
    def kfn(q_hbm, k_hbm, src_hbm, dst_hbm, ex_hbm, den_hbm,
            den_sh, srcv, dstv, qbuf, kbuf, exbuf, sem1, sem2):
        c = lax.axis_index("c")
        s = lax.axis_index("s")
        wid = c * NS + s
        zero16 = jnp.zeros((16,), f32)
        for r in range(16):
            exbuf[r, :] = zero16

        def zcopy(i, carry):
            pltpu.sync_copy(exbuf,
                            den_sh.at[pl.ds(s * ROWS_W + i * 16, 16)])
            return carry

        lax.fori_loop(0, ROWS_W // 16, zcopy, 0)
        plsc.subcore_barrier()

        lanes = lax.iota(i32, 16)

        def chunk(i, carry):
            eb = wid * EA_PER_W + i * L
            pltpu.sync_copy(src_hbm.at[pl.ds(eb, L)], srcv)
            pltpu.sync_copy(dst_hbm.at[pl.ds(eb, L)], dstv)
            cp1 = pltpu.async_copy(q_hbm.at[dstv], qbuf, sem1)
            cp2 = pltpu.async_copy(k_hbm.at[srcv], kbuf, sem2)
            cp1.wait()
            cp2.wait()
            for hh in range(H):
                acc = jnp.zeros((16,), f32)
                for d0 in range(C):
                    col = jnp.full((16,), hh * C + d0, i32)
                    acc = acc + (plsc.load_gather(qbuf, [lanes, col]) *
                                 plsc.load_gather(kbuf, [lanes, col]))
                exh = jnp.exp(acc * 0.125)
                plsc.store_scatter(exbuf, [lanes, jnp.full((16,), hh, i32)],
                                   exh)
            pltpu.sync_copy(exbuf, ex_hbm.at[pl.ds(eb, L)])
            pltpu.sync_copy(exbuf, den_sh.at[dstv], add=True)
            return carry

        lax.fori_loop(0, CHUNKS_A, chunk, 0)
        plsc.subcore_barrier()
        pltpu.sync_copy(den_sh.at[pl.ds(s * ROWS_W, ROWS_W)],
                        den_hbm.at[pl.ds(c * NPAD + s * ROWS_W, ROWS_W)])

    return kfn(q, k, srcp, dstp)


def _sc_phase_b(vstack, srcp, dstp, ex, rden):
    mesh = plsc.VectorSubcoreMesh(core_axis_name="c", subcore_axis_name="s")

    @functools.partial(
        pl.kernel,
        mesh=mesh,
        out_type=jax.ShapeDtypeStruct((2 * NPAD, 128), f32),
        compiler_params=pltpu.CompilerParams(
            use_tc_tiling_on_sc=False, needs_layout_passes=False),
        scratch_types=[
            pltpu.VMEM_SHARED((NPAD, 128), f32),
            pltpu.VMEM((16,), i32),
            pltpu.VMEM((16,), i32),
            pltpu.VMEM((16, 128), f32),
            pltpu.VMEM((16, 128), f32),
            pltpu.VMEM((16, 16), f32),
            pltpu.VMEM((16, 16), f32),
            pltpu.VMEM((16, 16), f32),
            pltpu.SemaphoreType.DMA,
        ],
    )
    def kfn(v_hbm, src_hbm, dst_hbm, ex_hbm, rden_hbm, out_hbm,
            oacc, dstv, sidx, vbuf, stage, exv, rdenv, abuf, sem):
        c = lax.axis_index("c")
        s = lax.axis_index("s")
        zero16 = jnp.zeros((16,), f32)

        for r in range(16):
            for j in range(8):
                stage[r, pl.ds(16 * j, 16)] = zero16

        def zcopy(i, carry):
            pltpu.sync_copy(stage,
                            oacc.at[pl.ds(s * ROWS_W + i * 16, 16)])
            return carry

        lax.fori_loop(0, ROWS_W // 16, zcopy, 0)
        plsc.subcore_barrier()

        lanes = lax.iota(i32, 16)
        voff = c * NPAD
        h0col = c * 2

        def chunk(i, carry):
            eb = s * EB_PER_W + i * L
            pltpu.sync_copy(src_hbm.at[pl.ds(eb, L)], sidx)
            pltpu.sync_copy(dst_hbm.at[pl.ds(eb, L)], dstv)
            sidx[...] = sidx[...] + jnp.full((16,), voff, i32)
            cp1 = pltpu.async_copy(v_hbm.at[sidx], vbuf, sem)
            cp1.wait()
            pltpu.sync_copy(ex_hbm.at[pl.ds(eb, L)], exv)
            cp2 = pltpu.async_copy(rden_hbm.at[dstv], rdenv, sem)
            cp2.wait()
            for j in range(2):
                colv = jnp.full((16,), h0col + j, i32)
                aj = (plsc.load_gather(exv, [lanes, colv]) *
                      plsc.load_gather(rdenv, [lanes, colv]))
                plsc.store_scatter(abuf, [lanes, jnp.full((16,), j, i32)], aj)
            for e in range(16):
                arow = abuf[e, :]
                a0 = arow[0]
                a1 = arow[1]
                for j in range(8):
                    aa = a0 if j < 4 else a1
                    stage[e, pl.ds(16 * j, 16)] = (
                        vbuf[e, pl.ds(16 * j, 16)] * aa)
            pltpu.sync_copy(stage, oacc.at[dstv], add=True)
            return carry

        lax.fori_loop(0, CHUNKS_B, chunk, 0)
        plsc.subcore_barrier()
        pltpu.sync_copy(oacc.at[pl.ds(s * ROWS_W, ROWS_W)],
                        out_hbm.at[pl.ds(c * NPAD + s * ROWS_W, ROWS_W)])

    return kfn(vstack, srcp, dstp, ex, rden)


def kernel(x, edge_index, batch, W_emb, b_emb, Wq0, bq0, Wk0, bk0, Wv0, bv0,
           Ws0, bs0, Wq1, bq1, Wk1, bk1, Wv1, bv1, Ws1, bs1, W_fc, b_fc):
    x_pad = jnp.pad(x, ((0, NPAD - N), (0, 0)))
    pad_idx = jnp.full((EPAD - E,), N, i32)
    srcp = jnp.concatenate([edge_index[0].astype(i32), pad_idx])
    dstp = jnp.concatenate([edge_index[1].astype(i32), pad_idx])

    h = _mm(x_pad, W_emb, b_emb.reshape(1, -1))
    layers = [(Wq0, bq0, Wk0, bk0, Wv0, bv0, Ws0, bs0),
              (Wq1, bq1, Wk1, bk1, Wv1, bv1, Ws1, bs1)]
    for (Wq, bq, Wk, bk, Wv, bv, Ws, bs) in layers:
        W4 = jnp.concatenate([Wq, Wk, Wv, Ws], axis=1)
        b4 = jnp.concatenate([bq, bk, bv, bs]).reshape(1, -1)
        o = _mm(h, W4, b4)
        q = o[:, :256]
        kk = o[:, 256:512]
        vstack = jnp.concatenate([o[:, 512:640], o[:, 640:768]], axis=0)
        sproj = o[:, 768:]
        ex, den = _sc_phase_a(q, kk, srcp, dstp)
        rden = _rden(den)
        outflat = _sc_phase_b(vstack, srcp, dstp, ex, rden)
        h = _reluadd(outflat, sproj)

    out = _pool(h, batch.astype(i32).reshape(10, 1, 1000),
                W_fc, b_fc.reshape(1, 1))
    return out.reshape(G)


# speedup vs baseline: 9.8721x; 1.8102x over previous
import functools

import jax
import jax.numpy as jnp
from jax import lax
from jax.experimental import pallas as pl
from jax.experimental.pallas import tpu as pltpu
from jax.experimental.pallas import tpu_sc as plsc

N = 10000
E = 160000
D = 256
HID = 256
H = 4
C = 64
G = 16

NC = 2
NS = 16
L = 16
NW = NC * NS
NPAD = 10240
EPAD = 161792
NROWS = EPAD // 16 + 4
ROWS_W = NPAD // NS
EA_PER_W = EPAD // NW
CHUNKS_A = EA_PER_W // L
EB_PER_W = EPAD // NS
CHUNKS_B = EB_PER_W // L

f32 = jnp.float32
i32 = jnp.int32


def _mm(x, w, b, bm=1280):
    n, k = x.shape
    m = w.shape[1]

    def body(x_ref, w_ref, b_ref, o_ref):
        o_ref[...] = jnp.dot(x_ref[...], w_ref[...],
                             preferred_element_type=f32) + b_ref[...]

    return pl.pallas_call(
        body,
        grid=(n // bm,),
        in_specs=[pl.BlockSpec((bm, k), lambda i: (i, 0)),
                  pl.BlockSpec((k, m), lambda i: (0, 0)),
                  pl.BlockSpec((1, m), lambda i: (0, 0))],
        out_specs=pl.BlockSpec((bm, m), lambda i: (i, 0)),
        out_shape=jax.ShapeDtypeStruct((n, m), f32),
    )(x, w, b)


def _rden(den):
    def body(a_ref, b_ref, o_ref):
        o_ref[...] = 1.0 / (a_ref[...] + b_ref[...] + 1e-16)

    return pl.pallas_call(
        body,
        grid=(8,),
        in_specs=[pl.BlockSpec((1280, 16), lambda i: (i, 0)),
                  pl.BlockSpec((1280, 16), lambda i: (i + 8, 0))],
        out_specs=pl.BlockSpec((1280, 16), lambda i: (i, 0)),
        out_shape=jax.ShapeDtypeStruct((NPAD, 16), f32),
    )(den, den)


def _reluadd(outflat, sproj):
    def body(l_ref, r_ref, s_ref, o_ref):
        cat = jnp.concatenate([l_ref[...], r_ref[...]], axis=1)
        o_ref[...] = jnp.maximum(cat + s_ref[...], 0.0)

    return pl.pallas_call(
        body,
        grid=(8,),
        in_specs=[pl.BlockSpec((1280, 128), lambda i: (i, 0)),
                  pl.BlockSpec((1280, 128), lambda i: (i + 8, 0)),
                  pl.BlockSpec((1280, 256), lambda i: (i, 0))],
        out_specs=pl.BlockSpec((1280, 256), lambda i: (i, 0)),
        out_shape=jax.ShapeDtypeStruct((NPAD, 256), f32),
    )(outflat, outflat, sproj)


def _pool(h, bat3, w_fc, b_fc):
    def body(h_ref, b_ref, w_ref, bias_ref, o_ref, pacc, cacc):
        i = pl.program_id(0)

        @pl.when(i == 0)
        def _():
            pacc[...] = jnp.zeros_like(pacc)
            cacc[...] = jnp.zeros_like(cacc)

        bat = b_ref[0]
        gids = lax.broadcasted_iota(i32, (G, 1000), 0)
        oh = jnp.where(bat == gids, 1.0, 0.0)
        pacc[...] += jnp.dot(oh, h_ref[...], preferred_element_type=f32)
        cacc[...] += jnp.dot(oh, jnp.ones((1000, 128), f32),
                             preferred_element_type=f32)

        @pl.when(i == 9)
        def _():
            cnt = jnp.maximum(cacc[:, :1], 1.0)
            pooled = pacc[...] / cnt
            o_ref[...] = jnp.dot(pooled, w_ref[...],
                                 preferred_element_type=f32) + bias_ref[...]

    return pl.pallas_call(
        body,
        grid=(10,),
        in_specs=[pl.BlockSpec((1000, 256), lambda i: (i, 0)),
                  pl.BlockSpec((1, 1, 1000), lambda i: (i, 0, 0)),
                  pl.BlockSpec((256, 1), lambda i: (0, 0)),
                  pl.BlockSpec((1, 1), lambda i: (0, 0))],
        out_specs=pl.BlockSpec((G, 1), lambda i: (0, 0)),
        out_shape=jax.ShapeDtypeStruct((G, 1), f32),
        scratch_shapes=[pltpu.VMEM((G, 256), f32), pltpu.VMEM((G, 128), f32)],
    )(h, bat3, w_fc, b_fc)


_SC_PARAMS = dict(
    compiler_params=pltpu.CompilerParams(
        use_tc_tiling_on_sc=False, needs_layout_passes=False),
)


def _sc_phase_a(q, k, src3, dst3):
    mesh = plsc.VectorSubcoreMesh(core_axis_name="c", subcore_axis_name="s")

    @functools.partial(
        pl.kernel,
        mesh=mesh,
        out_type=(jax.ShapeDtypeStruct((EPAD + 64, 16), f32),
                  jax.ShapeDtypeStruct((2 * NPAD, 16), f32)),
        scratch_types=[
            pltpu.VMEM_SHARED((NPAD, 16), f32),
            [pltpu.VMEM((2, 16), i32)] * 2,
            [pltpu.VMEM((2, 16), i32)] * 2,
            [pltpu.VMEM((16, 256), f32)] * 2,
            [pltpu.VMEM((16, 256), f32)] * 2,
            [pltpu.VMEM((16, 16), f32)] * 2,
            [pltpu.VMEM((16,), i32)] * 2,
            pltpu.VMEM((16,), i32),
            [pltpu.SemaphoreType.DMA] * 2,
            [pltpu.SemaphoreType.DMA] * 2,
            [pltpu.SemaphoreType.DMA] * 2,
            [pltpu.SemaphoreType.DMA] * 2,
        ],
        **_SC_PARAMS,
    )
    def kfn(q_hbm, k_hbm, src_hbm, dst_hbm, ex_hbm, den_hbm,
            den_sh, srcslab, dstslab, qbuf, kbuf, exbuf, scidx, dumv,
            semq, semk, semex, semsc):
        c = lax.axis_index("c")
        s = lax.axis_index("s")
        wid = c * NS + s
        wrow = wid * CHUNKS_A
        wbase = wid * EA_PER_W
        zero16 = jnp.zeros((16,), f32)
        for b in range(2):
            for r in range(16):
                exbuf[b][r, :] = zero16

        def zcopy(i, carry):
            pltpu.sync_copy(exbuf[0],
                            den_sh.at[pl.ds(s * ROWS_W + i * 16, 16)])
            return carry

        lax.fori_loop(0, ROWS_W // 16, zcopy, 0)
        plsc.subcore_barrier()

        lanes = lax.iota(i32, 16)
        dumv[...] = jnp.full((16,), N, i32)

        for sl in range(2):
            pltpu.sync_copy(src_hbm.at[pl.ds(wrow + 2 * sl, 2)], srcslab[sl])
            pltpu.sync_copy(dst_hbm.at[pl.ds(wrow + 2 * sl, 2)], dstslab[sl])
        for b in range(2):
            scidx[b][...] = dumv[...]
            pltpu.async_copy(exbuf[b], ex_hbm.at[pl.ds(EPAD + 16 * b, 16)],
                             semex[b])
            pltpu.async_copy(exbuf[b], den_sh.at[scidx[b]], semsc[b],
                             add=True)
            pltpu.async_copy(q_hbm.at[dstslab[0].at[b]], qbuf[b], semq[b])
            pltpu.async_copy(k_hbm.at[srcslab[0].at[b]], kbuf[b], semk[b])

        def chunk(i, b, sl, so, cb):
            eb = wbase + i * L
            cur_dst = dstslab[sl].at[cb]
            cur_src = srcslab[sl].at[cb]
            pltpu.make_async_copy(q_hbm.at[cur_dst], qbuf[b], semq[b]).wait()
            pltpu.make_async_copy(k_hbm.at[cur_src], kbuf[b], semk[b]).wait()
            pltpu.make_async_copy(exbuf[b], ex_hbm.at[pl.ds(eb, L)],
                                  semex[b]).wait()
            pltpu.make_async_copy(exbuf[b], den_sh.at[scidx[b]],
                                  semsc[b]).wait()
            for hh in range(H):
                acc = jnp.zeros((16,), f32)
                for d0 in range(C):
                    col = jnp.full((16,), hh * C + d0, i32)
                    acc = acc + (plsc.load_gather(qbuf[b], [lanes, col]) *
                                 plsc.load_gather(kbuf[b], [lanes, col]))
                exh = jnp.exp(acc * 0.125)
                plsc.store_scatter(exbuf[b],
                                   [lanes, jnp.full((16,), hh, i32)], exh)
            scidx[b][...] = dstslab[sl][cb, :]
            pltpu.async_copy(exbuf[b], ex_hbm.at[pl.ds(eb, L)], semex[b])
            pltpu.async_copy(exbuf[b], den_sh.at[scidx[b]], semsc[b],
                             add=True)
            pltpu.async_copy(q_hbm.at[dstslab[so].at[cb]], qbuf[b], semq[b])
            pltpu.async_copy(k_hbm.at[srcslab[so].at[cb]], kbuf[b], semk[b])

        def quad(qi, carry):
            i0 = 4 * qi
            for pb in range(2):
                sl, so = pb, 1 - pb
                for cb in range(2):
                    chunk(i0 + 2 * pb + cb, cb, sl, so, cb)
                rb = wrow + i0 // 2 + 2 * pb + 4
                pltpu.sync_copy(src_hbm.at[pl.ds(rb, 2)], srcslab[sl])
                pltpu.sync_copy(dst_hbm.at[pl.ds(rb, 2)], dstslab[sl])
            return carry

        lax.fori_loop(0, CHUNKS_A // 4, quad, 0)

        for b in range(2):
            pltpu.make_async_copy(q_hbm.at[dstslab[0].at[b]], qbuf[b],
                                  semq[b]).wait()
            pltpu.make_async_copy(k_hbm.at[srcslab[0].at[b]], kbuf[b],
                                  semk[b]).wait()
            pltpu.make_async_copy(exbuf[b], ex_hbm.at[pl.ds(EPAD, 16)],
                                  semex[b]).wait()
            pltpu.make_async_copy(exbuf[b], den_sh.at[scidx[b]],
                                  semsc[b]).wait()
        plsc.subcore_barrier()
        pltpu.sync_copy(den_sh.at[pl.ds(s * ROWS_W, ROWS_W)],
                        den_hbm.at[pl.ds(c * NPAD + s * ROWS_W, ROWS_W)])

    return kfn(q, k, src3, dst3)


def _sc_phase_b(vstack, src3, dst3, ex, rden):
    mesh = plsc.VectorSubcoreMesh(core_axis_name="c", subcore_axis_name="s")

    @functools.partial(
        pl.kernel,
        mesh=mesh,
        out_type=jax.ShapeDtypeStruct((2 * NPAD, 128), f32),
        scratch_types=[
            pltpu.VMEM_SHARED((NPAD, 128), f32),
            [pltpu.VMEM((2, 16), i32)] * 2,
            [pltpu.VMEM((2, 16), i32)] * 2,
            [pltpu.VMEM((16, 128), f32)] * 2,
            [pltpu.VMEM((16, 128), f32)] * 2,
            [pltpu.VMEM((16, 16), f32)] * 2,
            [pltpu.VMEM((16, 16), f32)] * 2,
            pltpu.VMEM((16, 16), f32),
            [pltpu.VMEM((16,), i32)] * 2,
            [pltpu.VMEM((16,), i32)] * 2,
            pltpu.VMEM((16,), i32),
            [pltpu.SemaphoreType.DMA] * 2,
            [pltpu.SemaphoreType.DMA] * 2,
            [pltpu.SemaphoreType.DMA] * 2,
            [pltpu.SemaphoreType.DMA] * 2,
        ],
        **_SC_PARAMS,
    )
    def kfn(v_hbm, src_hbm, dst_hbm, ex_hbm, rden_hbm, out_hbm,
            oacc, srcslab, dstslab, vbuf, stage, exv, rdenv, abuf,
            sidx, scidx, dumv, semv, semr, seme, semsc):
        c = lax.axis_index("c")
        s = lax.axis_index("s")
        srow = s * CHUNKS_B
        sbase = s * EB_PER_W
        zero16 = jnp.zeros((16,), f32)
        for b in range(2):
            for r in range(16):
                for j in range(8):
                    stage[b][r, pl.ds(16 * j, 16)] = zero16

        def zcopy(i, carry):
            pltpu.sync_copy(stage[0],
                            oacc.at[pl.ds(s * ROWS_W + i * 16, 16)])
            return carry

        lax.fori_loop(0, ROWS_W // 16, zcopy, 0)
        plsc.subcore_barrier()

        lanes = lax.iota(i32, 16)
        dumv[...] = jnp.full((16,), N, i32)
        voff = c * NPAD
        h0col = c * 2

        for sl in range(2):
            pltpu.sync_copy(src_hbm.at[pl.ds(srow + 2 * sl, 2)], srcslab[sl])
            pltpu.sync_copy(dst_hbm.at[pl.ds(srow + 2 * sl, 2)], dstslab[sl])
        for b in range(2):
            scidx[b][...] = dumv[...]
            pltpu.async_copy(stage[b], oacc.at[scidx[b]], semsc[b], add=True)
            sidx[b][...] = srcslab[0][b, :] + jnp.full((16,), voff, i32)
            pltpu.async_copy(v_hbm.at[sidx[b]], vbuf[b], semv[b])
            pltpu.async_copy(rden_hbm.at[dstslab[0].at[b]], rdenv[b],
                             semr[b])
            pltpu.async_copy(ex_hbm.at[pl.ds(sbase + b * L, L)], exv[b],
                             seme[b])

        def chunk(i, b, sl, so, cb):
            eb = sbase + i * L
            cur_dst = dstslab[sl].at[cb]
            pltpu.make_async_copy(v_hbm.at[sidx[b]], vbuf[b], semv[b]).wait()
            pltpu.make_async_copy(rden_hbm.at[cur_dst], rdenv[b],
                                  semr[b]).wait()
            pltpu.make_async_copy(ex_hbm.at[pl.ds(eb, L)], exv[b],
                                  seme[b]).wait()
            pltpu.make_async_copy(stage[b], oacc.at[scidx[b]],
                                  semsc[b]).wait()
            for j in range(2):
                colv = jnp.full((16,), h0col + j, i32)
                aj = (plsc.load_gather(exv[b], [lanes, colv]) *
                      plsc.load_gather(rdenv[b], [lanes, colv]))
                plsc.store_scatter(abuf, [lanes, jnp.full((16,), j, i32)],
                                   aj)
            for e in range(16):
                arow = abuf[e, :]
                a0 = arow[0]
                a1 = arow[1]
                for j in range(8):
                    aa = a0 if j < 4 else a1
                    stage[b][e, pl.ds(16 * j, 16)] = (
                        vbuf[b][e, pl.ds(16 * j, 16)] * aa)
            scidx[b][...] = dstslab[sl][cb, :]
            pltpu.async_copy(stage[b], oacc.at[scidx[b]], semsc[b], add=True)
            sidx[b][...] = srcslab[so][cb, :] + jnp.full((16,), voff, i32)
            pltpu.async_copy(v_hbm.at[sidx[b]], vbuf[b], semv[b])
            pltpu.async_copy(rden_hbm.at[dstslab[so].at[cb]], rdenv[b],
                             semr[b])
            pltpu.async_copy(ex_hbm.at[pl.ds(eb + 2 * L, L)], exv[b],
                             seme[b])

        def quad(qi, carry):
            i0 = 4 * qi
            for pb in range(2):
                sl, so = pb, 1 - pb
                for cb in range(2):
                    chunk(i0 + 2 * pb + cb, cb, sl, so, cb)
                rb = srow + i0 // 2 + 2 * pb + 4
                pltpu.sync_copy(src_hbm.at[pl.ds(rb, 2)], srcslab[sl])
                pltpu.sync_copy(dst_hbm.at[pl.ds(rb, 2)], dstslab[sl])
            return carry

        lax.fori_loop(0, CHUNKS_B // 4, quad, 0)

        for b in range(2):
            pltpu.make_async_copy(v_hbm.at[sidx[b]], vbuf[b], semv[b]).wait()
            pltpu.make_async_copy(rden_hbm.at[dstslab[0].at[b]], rdenv[b],
                                  semr[b]).wait()
            pltpu.make_async_copy(ex_hbm.at[pl.ds(sbase, L)], exv[b],
                                  seme[b]).wait()
            pltpu.make_async_copy(stage[b], oacc.at[scidx[b]],
                                  semsc[b]).wait()
        plsc.subcore_barrier()
        pltpu.sync_copy(oacc.at[pl.ds(s * ROWS_W, ROWS_W)],
                        out_hbm.at[pl.ds(c * NPAD + s * ROWS_W, ROWS_W)])

    return kfn(vstack, src3, dst3, ex, rden)


def kernel(x, edge_index, batch, W_emb, b_emb, Wq0, bq0, Wk0, bk0, Wv0, bv0,
           Ws0, bs0, Wq1, bq1, Wk1, bk1, Wv1, bv1, Ws1, bs1, W_fc, b_fc):
    x_pad = jnp.pad(x, ((0, NPAD - N), (0, 0)))
    pad_idx = jnp.full((NROWS * 16 - E,), N, i32)
    src3 = jnp.concatenate([edge_index[0].astype(i32),
                            pad_idx]).reshape(NROWS, 16)
    dst3 = jnp.concatenate([edge_index[1].astype(i32),
                            pad_idx]).reshape(NROWS, 16)

    h = _mm(x_pad, W_emb, b_emb.reshape(1, -1))
    layers = [(Wq0, bq0, Wk0, bk0, Wv0, bv0, Ws0, bs0),
              (Wq1, bq1, Wk1, bk1, Wv1, bv1, Ws1, bs1)]
    for (Wq, bq, Wk, bk, Wv, bv, Ws, bs) in layers:
        W4 = jnp.concatenate([Wq, Wk, Wv, Ws], axis=1)
        b4 = jnp.concatenate([bq, bk, bv, bs]).reshape(1, -1)
        o = _mm(h, W4, b4)
        q = o[:, :256]
        kk = o[:, 256:512]
        vstack = jnp.concatenate([o[:, 512:640], o[:, 640:768]], axis=0)
        sproj = o[:, 768:]
        ex, den = _sc_phase_a(q, kk, src3, dst3)
        rden = _rden(den)
        outflat = _sc_phase_b(vstack, src3, dst3, ex, rden)
        h = _reluadd(outflat, sproj)

    out = _pool(h, batch.astype(i32).reshape(10, 1, 1000),
                W_fc, b_fc.reshape(1, 1))
    return out.reshape(G)


# speedup vs baseline: 9.9946x; 1.0124x over previous
import functools

import jax
import jax.numpy as jnp
from jax import lax
from jax.experimental import pallas as pl
from jax.experimental.pallas import tpu as pltpu
from jax.experimental.pallas import tpu_sc as plsc

N = 10000
E = 160000
D = 256
HID = 256
H = 4
C = 64
G = 16

NC = 2
NS = 16
L = 16
NW = NC * NS
NPAD = 10240
EPAD = 161792
NROWS = EPAD // 16 + 4
ROWS_W = NPAD // NS
EA_PER_W = EPAD // NW
CHUNKS_A = EA_PER_W // L
EB_PER_W = EPAD // NS
CHUNKS_B = EB_PER_W // L

f32 = jnp.float32
i32 = jnp.int32


def _mm(x, w, b, bm=1280):
    n, k = x.shape
    m = w.shape[1]

    def body(x_ref, w_ref, b_ref, o_ref):
        o_ref[...] = jnp.dot(x_ref[...], w_ref[...],
                             preferred_element_type=f32) + b_ref[...]

    return pl.pallas_call(
        body,
        grid=(n // bm,),
        in_specs=[pl.BlockSpec((bm, k), lambda i: (i, 0)),
                  pl.BlockSpec((k, m), lambda i: (0, 0)),
                  pl.BlockSpec((1, m), lambda i: (0, 0))],
        out_specs=pl.BlockSpec((bm, m), lambda i: (i, 0)),
        out_shape=jax.ShapeDtypeStruct((n, m), f32),
    )(x, w, b)


def _rden(den):
    def body(a_ref, b_ref, o_ref):
        o_ref[...] = 1.0 / (a_ref[...] + b_ref[...] + 1e-16)

    return pl.pallas_call(
        body,
        grid=(8,),
        in_specs=[pl.BlockSpec((1280, 16), lambda i: (i, 0)),
                  pl.BlockSpec((1280, 16), lambda i: (i + 8, 0))],
        out_specs=pl.BlockSpec((1280, 16), lambda i: (i, 0)),
        out_shape=jax.ShapeDtypeStruct((NPAD, 16), f32),
    )(den, den)


def _reluadd(outflat, sproj):
    def body(l_ref, r_ref, s_ref, o_ref):
        cat = jnp.concatenate([l_ref[...], r_ref[...]], axis=1)
        o_ref[...] = jnp.maximum(cat + s_ref[...], 0.0)

    return pl.pallas_call(
        body,
        grid=(8,),
        in_specs=[pl.BlockSpec((1280, 128), lambda i: (i, 0)),
                  pl.BlockSpec((1280, 128), lambda i: (i + 8, 0)),
                  pl.BlockSpec((1280, 256), lambda i: (i, 0))],
        out_specs=pl.BlockSpec((1280, 256), lambda i: (i, 0)),
        out_shape=jax.ShapeDtypeStruct((NPAD, 256), f32),
    )(outflat, outflat, sproj)


def _pool(h, bat3, w_fc, b_fc):
    def body(h_ref, b_ref, w_ref, bias_ref, o_ref, pacc, cacc):
        i = pl.program_id(0)

        @pl.when(i == 0)
        def _():
            pacc[...] = jnp.zeros_like(pacc)
            cacc[...] = jnp.zeros_like(cacc)

        bat = b_ref[0]
        gids = lax.broadcasted_iota(i32, (G, 1000), 0)
        oh = jnp.where(bat == gids, 1.0, 0.0)
        pacc[...] += jnp.dot(oh, h_ref[...], preferred_element_type=f32)
        cacc[...] += jnp.dot(oh, jnp.ones((1000, 128), f32),
                             preferred_element_type=f32)

        @pl.when(i == 9)
        def _():
            cnt = jnp.maximum(cacc[:, :1], 1.0)
            pooled = pacc[...] / cnt
            o_ref[...] = jnp.dot(pooled, w_ref[...],
                                 preferred_element_type=f32) + bias_ref[...]

    return pl.pallas_call(
        body,
        grid=(10,),
        in_specs=[pl.BlockSpec((1000, 256), lambda i: (i, 0)),
                  pl.BlockSpec((1, 1, 1000), lambda i: (i, 0, 0)),
                  pl.BlockSpec((256, 1), lambda i: (0, 0)),
                  pl.BlockSpec((1, 1), lambda i: (0, 0))],
        out_specs=pl.BlockSpec((G, 1), lambda i: (0, 0)),
        out_shape=jax.ShapeDtypeStruct((G, 1), f32),
        scratch_shapes=[pltpu.VMEM((G, 256), f32), pltpu.VMEM((G, 128), f32)],
    )(h, bat3, w_fc, b_fc)


_SC_PARAMS = dict(
    compiler_params=pltpu.CompilerParams(
        use_tc_tiling_on_sc=False, needs_layout_passes=False),
)


def _sc_phase_a(q, k, src3, dst3):
    mesh = plsc.VectorSubcoreMesh(core_axis_name="c", subcore_axis_name="s")

    @functools.partial(
        pl.kernel,
        mesh=mesh,
        out_type=(jax.ShapeDtypeStruct((EPAD + 64, 16), f32),
                  jax.ShapeDtypeStruct((2 * NPAD, 16), f32)),
        scratch_types=[
            pltpu.VMEM_SHARED((NPAD, 16), f32),
            [pltpu.VMEM((2, 16), i32)] * 2,
            [pltpu.VMEM((2, 16), i32)] * 2,
            [pltpu.VMEM((16, 256), f32)] * 4,
            [pltpu.VMEM((16, 256), f32)] * 4,
            [pltpu.VMEM((16, 16), f32)] * 2,
            [pltpu.VMEM((16,), i32)] * 2,
            pltpu.VMEM((16,), i32),
            [pltpu.SemaphoreType.DMA] * 4,
            [pltpu.SemaphoreType.DMA] * 4,
            [pltpu.SemaphoreType.DMA] * 2,
            [pltpu.SemaphoreType.DMA] * 2,
        ],
        **_SC_PARAMS,
    )
    def kfn(q_hbm, k_hbm, src_hbm, dst_hbm, ex_hbm, den_hbm,
            den_sh, srcslab, dstslab, qbuf, kbuf, exbuf, scidx, dumv,
            semq, semk, semex, semsc):
        c = lax.axis_index("c")
        s = lax.axis_index("s")
        wid = c * NS + s
        wrow = wid * CHUNKS_A
        wbase = wid * EA_PER_W
        zero16 = jnp.zeros((16,), f32)
        for b in range(2):
            for r in range(16):
                exbuf[b][r, :] = zero16

        def zcopy(i, carry):
            pltpu.sync_copy(exbuf[0],
                            den_sh.at[pl.ds(s * ROWS_W + i * 16, 16)])
            return carry

        lax.fori_loop(0, ROWS_W // 16, zcopy, 0)
        plsc.subcore_barrier()

        lanes = lax.iota(i32, 16)
        dumv[...] = jnp.full((16,), N, i32)

        for sl in range(2):
            pltpu.sync_copy(src_hbm.at[pl.ds(wrow + 2 * sl, 2)], srcslab[sl])
            pltpu.sync_copy(dst_hbm.at[pl.ds(wrow + 2 * sl, 2)], dstslab[sl])
        for b in range(2):
            scidx[b][...] = dumv[...]
            pltpu.async_copy(exbuf[b], ex_hbm.at[pl.ds(EPAD + 16 * b, 16)],
                             semex[b])
            pltpu.async_copy(exbuf[b], den_sh.at[scidx[b]], semsc[b],
                             add=True)
        for sl in range(2):
            for cb in range(2):
                j = 2 * sl + cb
                pltpu.async_copy(q_hbm.at[dstslab[sl].at[cb]], qbuf[j],
                                 semq[j])
                pltpu.async_copy(k_hbm.at[srcslab[sl].at[cb]], kbuf[j],
                                 semk[j])

        def chunk(i, j, b, sl, cb):
            eb = wbase + i * L
            cur_dst = dstslab[sl].at[cb]
            cur_src = srcslab[sl].at[cb]
            pltpu.make_async_copy(q_hbm.at[cur_dst], qbuf[j], semq[j]).wait()
            pltpu.make_async_copy(k_hbm.at[cur_src], kbuf[j], semk[j]).wait()
            pltpu.make_async_copy(exbuf[b], ex_hbm.at[pl.ds(eb, L)],
                                  semex[b]).wait()
            pltpu.make_async_copy(exbuf[b], den_sh.at[scidx[b]],
                                  semsc[b]).wait()
            for hh in range(H):
                accs = [jnp.zeros((16,), f32) for _ in range(4)]
                for d0 in range(C):
                    col = jnp.full((16,), hh * C + d0, i32)
                    accs[d0 % 4] = accs[d0 % 4] + (
                        plsc.load_gather(qbuf[j], [lanes, col]) *
                        plsc.load_gather(kbuf[j], [lanes, col]))
                acc = (accs[0] + accs[1]) + (accs[2] + accs[3])
                exh = jnp.exp(acc * 0.125)
                plsc.store_scatter(exbuf[b],
                                   [lanes, jnp.full((16,), hh, i32)], exh)
            scidx[b][...] = dstslab[sl][cb, :]
            pltpu.async_copy(exbuf[b], ex_hbm.at[pl.ds(eb, L)], semex[b])
            pltpu.async_copy(exbuf[b], den_sh.at[scidx[b]], semsc[b],
                             add=True)

        def quad(qi, carry):
            i0 = 4 * qi
            for pb in range(2):
                sl = pb
                for cb in range(2):
                    i = i0 + 2 * pb + cb
                    chunk(i, 2 * pb + cb, cb, sl, cb)
                rb = wrow + i0 // 2 + 2 * pb + 4
                pltpu.sync_copy(src_hbm.at[pl.ds(rb, 2)], srcslab[sl])
                pltpu.sync_copy(dst_hbm.at[pl.ds(rb, 2)], dstslab[sl])
                for cb in range(2):
                    j = 2 * pb + cb
                    pltpu.async_copy(q_hbm.at[dstslab[sl].at[cb]], qbuf[j],
                                     semq[j])
                    pltpu.async_copy(k_hbm.at[srcslab[sl].at[cb]], kbuf[j],
                                     semk[j])
            return carry

        lax.fori_loop(0, CHUNKS_A // 4, quad, 0)

        for j in range(4):
            pltpu.make_async_copy(q_hbm.at[dstslab[j // 2].at[j % 2]],
                                  qbuf[j], semq[j]).wait()
            pltpu.make_async_copy(k_hbm.at[srcslab[j // 2].at[j % 2]],
                                  kbuf[j], semk[j]).wait()
        for b in range(2):
            pltpu.make_async_copy(exbuf[b], ex_hbm.at[pl.ds(EPAD, 16)],
                                  semex[b]).wait()
            pltpu.make_async_copy(exbuf[b], den_sh.at[scidx[b]],
                                  semsc[b]).wait()
        plsc.subcore_barrier()
        pltpu.sync_copy(den_sh.at[pl.ds(s * ROWS_W, ROWS_W)],
                        den_hbm.at[pl.ds(c * NPAD + s * ROWS_W, ROWS_W)])

    return kfn(q, k, src3, dst3)


def _sc_phase_b(vstack, src3, dst3, ex, rden):
    mesh = plsc.VectorSubcoreMesh(core_axis_name="c", subcore_axis_name="s")

    @functools.partial(
        pl.kernel,
        mesh=mesh,
        out_type=jax.ShapeDtypeStruct((2 * NPAD, 128), f32),
        scratch_types=[
            pltpu.VMEM_SHARED((NPAD, 128), f32),
            [pltpu.VMEM((2, 16), i32)] * 2,
            [pltpu.VMEM((2, 16), i32)] * 2,
            [pltpu.VMEM((16, 128), f32)] * 4,
            [pltpu.VMEM((16, 128), f32)] * 2,
            [pltpu.VMEM((16, 16), f32)] * 4,
            [pltpu.VMEM((16, 16), f32)] * 4,
            pltpu.VMEM((16, 16), f32),
            [pltpu.VMEM((16,), i32)] * 4,
            [pltpu.VMEM((16,), i32)] * 2,
            pltpu.VMEM((16,), i32),
            [pltpu.SemaphoreType.DMA] * 4,
            [pltpu.SemaphoreType.DMA] * 4,
            [pltpu.SemaphoreType.DMA] * 4,
            [pltpu.SemaphoreType.DMA] * 2,
        ],
        **_SC_PARAMS,
    )
    def kfn(v_hbm, src_hbm, dst_hbm, ex_hbm, rden_hbm, out_hbm,
            oacc, srcslab, dstslab, vbuf, stage, exv, rdenv, abuf,
            sidx, scidx, dumv, semv, semr, seme, semsc):
        c = lax.axis_index("c")
        s = lax.axis_index("s")
        srow = s * CHUNKS_B
        sbase = s * EB_PER_W
        zero16 = jnp.zeros((16,), f32)
        for b in range(2):
            for r in range(16):
                for j in range(8):
                    stage[b][r, pl.ds(16 * j, 16)] = zero16

        def zcopy(i, carry):
            pltpu.sync_copy(stage[0],
                            oacc.at[pl.ds(s * ROWS_W + i * 16, 16)])
            return carry

        lax.fori_loop(0, ROWS_W // 16, zcopy, 0)
        plsc.subcore_barrier()

        lanes = lax.iota(i32, 16)
        dumv[...] = jnp.full((16,), N, i32)
        voff = c * NPAD
        h0col = c * 2

        for sl in range(2):
            pltpu.sync_copy(src_hbm.at[pl.ds(srow + 2 * sl, 2)], srcslab[sl])
            pltpu.sync_copy(dst_hbm.at[pl.ds(srow + 2 * sl, 2)], dstslab[sl])
        for b in range(2):
            scidx[b][...] = dumv[...]
            pltpu.async_copy(stage[b], oacc.at[scidx[b]], semsc[b], add=True)
        for sl in range(2):
            for cb in range(2):
                j = 2 * sl + cb
                sidx[j][...] = srcslab[sl][cb, :] + jnp.full((16,), voff,
                                                            i32)
                pltpu.async_copy(v_hbm.at[sidx[j]], vbuf[j], semv[j])
                pltpu.async_copy(rden_hbm.at[dstslab[sl].at[cb]], rdenv[j],
                                 semr[j])
                pltpu.async_copy(ex_hbm.at[pl.ds(sbase + j * L, L)], exv[j],
                                 seme[j])

        def chunk(i, j, b, sl, cb):
            eb = sbase + i * L
            cur_dst = dstslab[sl].at[cb]
            pltpu.make_async_copy(v_hbm.at[sidx[j]], vbuf[j], semv[j]).wait()
            pltpu.make_async_copy(rden_hbm.at[cur_dst], rdenv[j],
                                  semr[j]).wait()
            pltpu.make_async_copy(ex_hbm.at[pl.ds(eb, L)], exv[j],
                                  seme[j]).wait()
            pltpu.make_async_copy(stage[b], oacc.at[scidx[b]],
                                  semsc[b]).wait()
            for jj in range(2):
                colv = jnp.full((16,), h0col + jj, i32)
                aj = (plsc.load_gather(exv[j], [lanes, colv]) *
                      plsc.load_gather(rdenv[j], [lanes, colv]))
                plsc.store_scatter(abuf, [lanes, jnp.full((16,), jj, i32)],
                                   aj)
            for e in range(16):
                arow = abuf[e, :]
                a0 = arow[0]
                a1 = arow[1]
                for jj in range(8):
                    aa = a0 if jj < 4 else a1
                    stage[b][e, pl.ds(16 * jj, 16)] = (
                        vbuf[j][e, pl.ds(16 * jj, 16)] * aa)
            scidx[b][...] = dstslab[sl][cb, :]
            pltpu.async_copy(stage[b], oacc.at[scidx[b]], semsc[b], add=True)

        def quad(qi, carry):
            i0 = 4 * qi
            for pb in range(2):
                sl = pb
                for cb in range(2):
                    i = i0 + 2 * pb + cb
                    chunk(i, 2 * pb + cb, cb, sl, cb)
                rb = srow + i0 // 2 + 2 * pb + 4
                pltpu.sync_copy(src_hbm.at[pl.ds(rb, 2)], srcslab[sl])
                pltpu.sync_copy(dst_hbm.at[pl.ds(rb, 2)], dstslab[sl])
                for cb in range(2):
                    j = 2 * pb + cb
                    i4 = i0 + 2 * pb + cb + 4
                    sidx[j][...] = srcslab[sl][cb, :] + jnp.full((16,), voff,
                                                                i32)
                    pltpu.async_copy(v_hbm.at[sidx[j]], vbuf[j], semv[j])
                    pltpu.async_copy(rden_hbm.at[dstslab[sl].at[cb]],
                                     rdenv[j], semr[j])
                    pltpu.async_copy(ex_hbm.at[pl.ds(sbase + i4 * L, L)],
                                     exv[j], seme[j])
            return carry

        lax.fori_loop(0, CHUNKS_B // 4, quad, 0)

        for j in range(4):
            pltpu.make_async_copy(v_hbm.at[sidx[j]], vbuf[j], semv[j]).wait()
            pltpu.make_async_copy(rden_hbm.at[dstslab[j // 2].at[j % 2]],
                                  rdenv[j], semr[j]).wait()
            pltpu.make_async_copy(ex_hbm.at[pl.ds(sbase, L)], exv[j],
                                  seme[j]).wait()
        for b in range(2):
            pltpu.make_async_copy(stage[b], oacc.at[scidx[b]],
                                  semsc[b]).wait()
        plsc.subcore_barrier()
        pltpu.sync_copy(oacc.at[pl.ds(s * ROWS_W, ROWS_W)],
                        out_hbm.at[pl.ds(c * NPAD + s * ROWS_W, ROWS_W)])

    return kfn(vstack, src3, dst3, ex, rden)


def kernel(x, edge_index, batch, W_emb, b_emb, Wq0, bq0, Wk0, bk0, Wv0, bv0,
           Ws0, bs0, Wq1, bq1, Wk1, bk1, Wv1, bv1, Ws1, bs1, W_fc, b_fc):
    x_pad = jnp.pad(x, ((0, NPAD - N), (0, 0)))
    pad_idx = jnp.full((NROWS * 16 - E,), N, i32)
    src3 = jnp.concatenate([edge_index[0].astype(i32),
                            pad_idx]).reshape(NROWS, 16)
    dst3 = jnp.concatenate([edge_index[1].astype(i32),
                            pad_idx]).reshape(NROWS, 16)

    h = _mm(x_pad, W_emb, b_emb.reshape(1, -1))
    layers = [(Wq0, bq0, Wk0, bk0, Wv0, bv0, Ws0, bs0),
              (Wq1, bq1, Wk1, bk1, Wv1, bv1, Ws1, bs1)]
    for (Wq, bq, Wk, bk, Wv, bv, Ws, bs) in layers:
        W4 = jnp.concatenate([Wq, Wk, Wv, Ws], axis=1)
        b4 = jnp.concatenate([bq, bk, bv, bs]).reshape(1, -1)
        o = _mm(h, W4, b4)
        q = o[:, :256]
        kk = o[:, 256:512]
        vstack = jnp.concatenate([o[:, 512:640], o[:, 640:768]], axis=0)
        sproj = o[:, 768:]
        ex, den = _sc_phase_a(q, kk, src3, dst3)
        rden = _rden(den)
        outflat = _sc_phase_b(vstack, src3, dst3, ex, rden)
        h = _reluadd(outflat, sproj)

    out = _pool(h, batch.astype(i32).reshape(10, 1, 1000),
                W_fc, b_fc.reshape(1, 1))
    return out.reshape(G)


# speedup vs baseline: 15.8474x; 1.5856x over previous
import functools

import jax
import jax.numpy as jnp
from jax import lax
from jax.experimental import pallas as pl
from jax.experimental.pallas import tpu as pltpu
from jax.experimental.pallas import tpu_sc as plsc

N = 10000
E = 160000
D = 256
HID = 256
H = 4
C = 64
G = 16

NC = 2
NS = 16
L = 16
NW = NC * NS
NPAD = 10240
EPAD = 161792
NROWS = EPAD // 16 + 4
ROWS_W = NPAD // NS
EA_PER_W = EPAD // NW
CHUNKS_A = EA_PER_W // L
EB_PER_W = EPAD // NS
CHUNKS_B = EB_PER_W // L

f32 = jnp.float32
i32 = jnp.int32


def _mm(x, w, b, bm=1280):
    n, k = x.shape
    m = w.shape[1]

    def body(x_ref, w_ref, b_ref, o_ref):
        o_ref[...] = jnp.dot(x_ref[...], w_ref[...],
                             preferred_element_type=f32) + b_ref[...]

    return pl.pallas_call(
        body,
        grid=(n // bm,),
        in_specs=[pl.BlockSpec((bm, k), lambda i: (i, 0)),
                  pl.BlockSpec((k, m), lambda i: (0, 0)),
                  pl.BlockSpec((1, m), lambda i: (0, 0))],
        out_specs=pl.BlockSpec((bm, m), lambda i: (i, 0)),
        out_shape=jax.ShapeDtypeStruct((n, m), f32),
    )(x, w, b)


def _rden(den):
    def body(a_ref, b_ref, o_ref):
        o_ref[...] = 1.0 / (a_ref[...] + b_ref[...] + 1e-16)

    return pl.pallas_call(
        body,
        grid=(8,),
        in_specs=[pl.BlockSpec((1280, 16), lambda i: (i, 0)),
                  pl.BlockSpec((1280, 16), lambda i: (i + 8, 0))],
        out_specs=pl.BlockSpec((1280, 16), lambda i: (i, 0)),
        out_shape=jax.ShapeDtypeStruct((NPAD, 16), f32),
    )(den, den)


def _reluadd(outflat, sproj):
    def body(l_ref, r_ref, s_ref, o_ref):
        cat = jnp.concatenate([l_ref[...], r_ref[...]], axis=1)
        o_ref[...] = jnp.maximum(cat + s_ref[...], 0.0)

    return pl.pallas_call(
        body,
        grid=(8,),
        in_specs=[pl.BlockSpec((1280, 128), lambda i: (i, 0)),
                  pl.BlockSpec((1280, 128), lambda i: (i + 8, 0)),
                  pl.BlockSpec((1280, 256), lambda i: (i, 0))],
        out_specs=pl.BlockSpec((1280, 256), lambda i: (i, 0)),
        out_shape=jax.ShapeDtypeStruct((NPAD, 256), f32),
    )(outflat, outflat, sproj)


def _pool(h, bat3, w_fc, b_fc):
    def body(h_ref, b_ref, w_ref, bias_ref, o_ref, pacc, cacc):
        i = pl.program_id(0)

        @pl.when(i == 0)
        def _():
            pacc[...] = jnp.zeros_like(pacc)
            cacc[...] = jnp.zeros_like(cacc)

        bat = b_ref[0]
        gids = lax.broadcasted_iota(i32, (G, 1000), 0)
        oh = jnp.where(bat == gids, 1.0, 0.0)
        pacc[...] += jnp.dot(oh, h_ref[...], preferred_element_type=f32)
        cacc[...] += jnp.dot(oh, jnp.ones((1000, 128), f32),
                             preferred_element_type=f32)

        @pl.when(i == 9)
        def _():
            cnt = jnp.maximum(cacc[:, :1], 1.0)
            pooled = pacc[...] / cnt
            o_ref[...] = jnp.dot(pooled, w_ref[...],
                                 preferred_element_type=f32) + bias_ref[...]

    return pl.pallas_call(
        body,
        grid=(10,),
        in_specs=[pl.BlockSpec((1000, 256), lambda i: (i, 0)),
                  pl.BlockSpec((1, 1, 1000), lambda i: (i, 0, 0)),
                  pl.BlockSpec((256, 1), lambda i: (0, 0)),
                  pl.BlockSpec((1, 1), lambda i: (0, 0))],
        out_specs=pl.BlockSpec((G, 1), lambda i: (0, 0)),
        out_shape=jax.ShapeDtypeStruct((G, 1), f32),
        scratch_shapes=[pltpu.VMEM((G, 256), f32), pltpu.VMEM((G, 128), f32)],
    )(h, bat3, w_fc, b_fc)


_SC_PARAMS = dict(
    compiler_params=pltpu.CompilerParams(
        use_tc_tiling_on_sc=False, needs_layout_passes=False),
)


def _sc_phase_a(q, k, src3, dst3):
    mesh = plsc.VectorSubcoreMesh(core_axis_name="c", subcore_axis_name="s")

    @functools.partial(
        pl.kernel,
        mesh=mesh,
        out_type=(jax.ShapeDtypeStruct((EPAD + 64, 16), f32),
                  jax.ShapeDtypeStruct((2 * NPAD, 16), f32)),
        scratch_types=[
            pltpu.VMEM_SHARED((NPAD, 16), f32),
            [pltpu.VMEM((2, 16), i32)] * 2,
            [pltpu.VMEM((2, 16), i32)] * 2,
            [pltpu.VMEM((16, 256), f32)] * 4,
            [pltpu.VMEM((16, 256), f32)] * 4,
            [pltpu.VMEM((16, 16), f32)] * 2,
            [pltpu.VMEM((16,), i32)] * 2,
            pltpu.VMEM((16,), i32),
            pltpu.VMEM((256, 16), i32),
            [pltpu.SemaphoreType.DMA] * 4,
            [pltpu.SemaphoreType.DMA] * 4,
            [pltpu.SemaphoreType.DMA] * 2,
            [pltpu.SemaphoreType.DMA] * 2,
        ],
        **_SC_PARAMS,
    )
    def kfn(q_hbm, k_hbm, src_hbm, dst_hbm, ex_hbm, den_hbm,
            den_sh, srcslab, dstslab, qbuf, kbuf, exbuf, scidx, dumv,
            coltab, semq, semk, semex, semsc):
        c = lax.axis_index("c")
        s = lax.axis_index("s")
        wid = c * NS + s
        wrow = wid * CHUNKS_A
        wbase = wid * EA_PER_W
        zero16 = jnp.zeros((16,), f32)
        for b in range(2):
            for r in range(16):
                exbuf[b][r, :] = zero16

        def zcopy(i, carry):
            pltpu.sync_copy(exbuf[0],
                            den_sh.at[pl.ds(s * ROWS_W + i * 16, 16)])
            return carry

        lax.fori_loop(0, ROWS_W // 16, zcopy, 0)
        plsc.subcore_barrier()

        lanes = lax.iota(i32, 16)
        dumv[...] = jnp.full((16,), N, i32)

        def colrow(r, carry):
            rv = jnp.full((16,), r, i32)
            coltab[r, :] = (jnp.bitwise_and(rv + lanes, C - 1) +
                            jnp.bitwise_and(rv, 192))
            return carry

        lax.fori_loop(0, 256, colrow, 0)

        for sl in range(2):
            pltpu.sync_copy(src_hbm.at[pl.ds(wrow + 2 * sl, 2)], srcslab[sl])
            pltpu.sync_copy(dst_hbm.at[pl.ds(wrow + 2 * sl, 2)], dstslab[sl])
        for b in range(2):
            scidx[b][...] = dumv[...]
            pltpu.async_copy(exbuf[b], ex_hbm.at[pl.ds(EPAD + 16 * b, 16)],
                             semex[b])
            pltpu.async_copy(exbuf[b], den_sh.at[scidx[b]], semsc[b],
                             add=True)
        for sl in range(2):
            for cb in range(2):
                j = 2 * sl + cb
                pltpu.async_copy(q_hbm.at[dstslab[sl].at[cb]], qbuf[j],
                                 semq[j])
                pltpu.async_copy(k_hbm.at[srcslab[sl].at[cb]], kbuf[j],
                                 semk[j])

        def chunk(i, j, b, sl, cb):
            eb = wbase + i * L
            cur_dst = dstslab[sl].at[cb]
            cur_src = srcslab[sl].at[cb]
            pltpu.make_async_copy(q_hbm.at[cur_dst], qbuf[j], semq[j]).wait()
            pltpu.make_async_copy(k_hbm.at[cur_src], kbuf[j], semk[j]).wait()
            pltpu.make_async_copy(exbuf[b], ex_hbm.at[pl.ds(eb, L)],
                                  semex[b]).wait()
            pltpu.make_async_copy(exbuf[b], den_sh.at[scidx[b]],
                                  semsc[b]).wait()
            for hh in range(H):
                accs = [jnp.zeros((16,), f32) for _ in range(4)]
                for d0 in range(C):
                    col = coltab[hh * C + d0, :]
                    accs[d0 % 4] = accs[d0 % 4] + (
                        plsc.load_gather(qbuf[j], [lanes, col]) *
                        plsc.load_gather(kbuf[j], [lanes, col]))
                acc = (accs[0] + accs[1]) + (accs[2] + accs[3])
                exh = jnp.exp(acc * 0.125)
                plsc.store_scatter(exbuf[b],
                                   [lanes, jnp.full((16,), hh, i32)], exh)
            scidx[b][...] = dstslab[sl][cb, :]
            pltpu.async_copy(exbuf[b], ex_hbm.at[pl.ds(eb, L)], semex[b])
            pltpu.async_copy(exbuf[b], den_sh.at[scidx[b]], semsc[b],
                             add=True)

        def quad(qi, carry):
            i0 = 4 * qi
            for pb in range(2):
                sl = pb
                for cb in range(2):
                    i = i0 + 2 * pb + cb
                    chunk(i, 2 * pb + cb, cb, sl, cb)
                rb = wrow + i0 // 2 + 2 * pb + 4
                pltpu.sync_copy(src_hbm.at[pl.ds(rb, 2)], srcslab[sl])
                pltpu.sync_copy(dst_hbm.at[pl.ds(rb, 2)], dstslab[sl])
                for cb in range(2):
                    j = 2 * pb + cb
                    pltpu.async_copy(q_hbm.at[dstslab[sl].at[cb]], qbuf[j],
                                     semq[j])
                    pltpu.async_copy(k_hbm.at[srcslab[sl].at[cb]], kbuf[j],
                                     semk[j])
            return carry

        lax.fori_loop(0, CHUNKS_A // 4, quad, 0)

        for j in range(4):
            pltpu.make_async_copy(q_hbm.at[dstslab[j // 2].at[j % 2]],
                                  qbuf[j], semq[j]).wait()
            pltpu.make_async_copy(k_hbm.at[srcslab[j // 2].at[j % 2]],
                                  kbuf[j], semk[j]).wait()
        for b in range(2):
            pltpu.make_async_copy(exbuf[b], ex_hbm.at[pl.ds(EPAD, 16)],
                                  semex[b]).wait()
            pltpu.make_async_copy(exbuf[b], den_sh.at[scidx[b]],
                                  semsc[b]).wait()
        plsc.subcore_barrier()
        pltpu.sync_copy(den_sh.at[pl.ds(s * ROWS_W, ROWS_W)],
                        den_hbm.at[pl.ds(c * NPAD + s * ROWS_W, ROWS_W)])

    return kfn(q, k, src3, dst3)


def _sc_phase_b(vstack, src3, dst3, ex, rden):
    mesh = plsc.VectorSubcoreMesh(core_axis_name="c", subcore_axis_name="s")

    @functools.partial(
        pl.kernel,
        mesh=mesh,
        out_type=jax.ShapeDtypeStruct((2 * NPAD, 128), f32),
        scratch_types=[
            pltpu.VMEM_SHARED((NPAD, 128), f32),
            [pltpu.VMEM((2, 16), i32)] * 2,
            [pltpu.VMEM((2, 16), i32)] * 2,
            [pltpu.VMEM((16, 128), f32)] * 4,
            [pltpu.VMEM((16, 128), f32)] * 2,
            [pltpu.VMEM((16, 16), f32)] * 4,
            [pltpu.VMEM((16, 16), f32)] * 4,
            pltpu.VMEM((16, 16), f32),
            [pltpu.VMEM((16,), i32)] * 4,
            [pltpu.VMEM((16,), i32)] * 2,
            pltpu.VMEM((16,), i32),
            [pltpu.SemaphoreType.DMA] * 4,
            [pltpu.SemaphoreType.DMA] * 4,
            [pltpu.SemaphoreType.DMA] * 4,
            [pltpu.SemaphoreType.DMA] * 2,
        ],
        **_SC_PARAMS,
    )
    def kfn(v_hbm, src_hbm, dst_hbm, ex_hbm, rden_hbm, out_hbm,
            oacc, srcslab, dstslab, vbuf, stage, exv, rdenv, abuf,
            sidx, scidx, dumv, semv, semr, seme, semsc):
        c = lax.axis_index("c")
        s = lax.axis_index("s")
        srow = s * CHUNKS_B
        sbase = s * EB_PER_W
        zero16 = jnp.zeros((16,), f32)
        for b in range(2):
            for r in range(16):
                for j in range(8):
                    stage[b][r, pl.ds(16 * j, 16)] = zero16

        def zcopy(i, carry):
            pltpu.sync_copy(stage[0],
                            oacc.at[pl.ds(s * ROWS_W + i * 16, 16)])
            return carry

        lax.fori_loop(0, ROWS_W // 16, zcopy, 0)
        plsc.subcore_barrier()

        lanes = lax.iota(i32, 16)
        dumv[...] = jnp.full((16,), N, i32)
        voff = c * NPAD
        h0col = c * 2

        for sl in range(2):
            pltpu.sync_copy(src_hbm.at[pl.ds(srow + 2 * sl, 2)], srcslab[sl])
            pltpu.sync_copy(dst_hbm.at[pl.ds(srow + 2 * sl, 2)], dstslab[sl])
        for b in range(2):
            scidx[b][...] = dumv[...]
            pltpu.async_copy(stage[b], oacc.at[scidx[b]], semsc[b], add=True)
        for sl in range(2):
            for cb in range(2):
                j = 2 * sl + cb
                sidx[j][...] = srcslab[sl][cb, :] + jnp.full((16,), voff,
                                                            i32)
                pltpu.async_copy(v_hbm.at[sidx[j]], vbuf[j], semv[j])
                pltpu.async_copy(rden_hbm.at[dstslab[sl].at[cb]], rdenv[j],
                                 semr[j])
                pltpu.async_copy(ex_hbm.at[pl.ds(sbase + j * L, L)], exv[j],
                                 seme[j])

        def chunk(i, j, b, sl, cb):
            eb = sbase + i * L
            cur_dst = dstslab[sl].at[cb]
            pltpu.make_async_copy(v_hbm.at[sidx[j]], vbuf[j], semv[j]).wait()
            pltpu.make_async_copy(rden_hbm.at[cur_dst], rdenv[j],
                                  semr[j]).wait()
            pltpu.make_async_copy(ex_hbm.at[pl.ds(eb, L)], exv[j],
                                  seme[j]).wait()
            pltpu.make_async_copy(stage[b], oacc.at[scidx[b]],
                                  semsc[b]).wait()
            for jj in range(2):
                colv = jnp.full((16,), h0col + jj, i32)
                aj = (plsc.load_gather(exv[j], [lanes, colv]) *
                      plsc.load_gather(rdenv[j], [lanes, colv]))
                plsc.store_scatter(abuf, [lanes, jnp.full((16,), jj, i32)],
                                   aj)
            for e in range(16):
                arow = abuf[e, :]
                a0 = arow[0]
                a1 = arow[1]
                for jj in range(8):
                    aa = a0 if jj < 4 else a1
                    stage[b][e, pl.ds(16 * jj, 16)] = (
                        vbuf[j][e, pl.ds(16 * jj, 16)] * aa)
            scidx[b][...] = dstslab[sl][cb, :]
            pltpu.async_copy(stage[b], oacc.at[scidx[b]], semsc[b], add=True)

        def quad(qi, carry):
            i0 = 4 * qi
            for pb in range(2):
                sl = pb
                for cb in range(2):
                    i = i0 + 2 * pb + cb
                    chunk(i, 2 * pb + cb, cb, sl, cb)
                rb = srow + i0 // 2 + 2 * pb + 4
                pltpu.sync_copy(src_hbm.at[pl.ds(rb, 2)], srcslab[sl])
                pltpu.sync_copy(dst_hbm.at[pl.ds(rb, 2)], dstslab[sl])
                for cb in range(2):
                    j = 2 * pb + cb
                    i4 = i0 + 2 * pb + cb + 4
                    sidx[j][...] = srcslab[sl][cb, :] + jnp.full((16,), voff,
                                                                i32)
                    pltpu.async_copy(v_hbm.at[sidx[j]], vbuf[j], semv[j])
                    pltpu.async_copy(rden_hbm.at[dstslab[sl].at[cb]],
                                     rdenv[j], semr[j])
                    pltpu.async_copy(ex_hbm.at[pl.ds(sbase + i4 * L, L)],
                                     exv[j], seme[j])
            return carry

        lax.fori_loop(0, CHUNKS_B // 4, quad, 0)

        for j in range(4):
            pltpu.make_async_copy(v_hbm.at[sidx[j]], vbuf[j], semv[j]).wait()
            pltpu.make_async_copy(rden_hbm.at[dstslab[j // 2].at[j % 2]],
                                  rdenv[j], semr[j]).wait()
            pltpu.make_async_copy(ex_hbm.at[pl.ds(sbase, L)], exv[j],
                                  seme[j]).wait()
        for b in range(2):
            pltpu.make_async_copy(stage[b], oacc.at[scidx[b]],
                                  semsc[b]).wait()
        plsc.subcore_barrier()
        pltpu.sync_copy(oacc.at[pl.ds(s * ROWS_W, ROWS_W)],
                        out_hbm.at[pl.ds(c * NPAD + s * ROWS_W, ROWS_W)])

    return kfn(vstack, src3, dst3, ex, rden)


def kernel(x, edge_index, batch, W_emb, b_emb, Wq0, bq0, Wk0, bk0, Wv0, bv0,
           Ws0, bs0, Wq1, bq1, Wk1, bk1, Wv1, bv1, Ws1, bs1, W_fc, b_fc):
    x_pad = jnp.pad(x, ((0, NPAD - N), (0, 0)))
    pad_idx = jnp.full((NROWS * 16 - E,), N, i32)
    src3 = jnp.concatenate([edge_index[0].astype(i32),
                            pad_idx]).reshape(NROWS, 16)
    dst3 = jnp.concatenate([edge_index[1].astype(i32),
                            pad_idx]).reshape(NROWS, 16)

    h = _mm(x_pad, W_emb, b_emb.reshape(1, -1))
    layers = [(Wq0, bq0, Wk0, bk0, Wv0, bv0, Ws0, bs0),
              (Wq1, bq1, Wk1, bk1, Wv1, bv1, Ws1, bs1)]
    for (Wq, bq, Wk, bk, Wv, bv, Ws, bs) in layers:
        W4 = jnp.concatenate([Wq, Wk, Wv, Ws], axis=1)
        b4 = jnp.concatenate([bq, bk, bv, bs]).reshape(1, -1)
        o = _mm(h, W4, b4)
        q = o[:, :256]
        kk = o[:, 256:512]
        vstack = jnp.concatenate([o[:, 512:640], o[:, 640:768]], axis=0)
        sproj = o[:, 768:]
        ex, den = _sc_phase_a(q, kk, src3, dst3)
        rden = _rden(den)
        outflat = _sc_phase_b(vstack, src3, dst3, ex, rden)
        h = _reluadd(outflat, sproj)

    out = _pool(h, batch.astype(i32).reshape(10, 1, 1000),
                W_fc, b_fc.reshape(1, 1))
    return out.reshape(G)


# speedup vs baseline: 16.9107x; 1.0671x over previous
import functools

import jax
import jax.numpy as jnp
from jax import lax
from jax.experimental import pallas as pl
from jax.experimental.pallas import tpu as pltpu
from jax.experimental.pallas import tpu_sc as plsc

N = 10000
E = 160000
D = 256
HID = 256
H = 4
C = 64
G = 16

NC = 2
NS = 16
L = 16
NW = NC * NS
NPAD = 10240
EPAD = 161792
NROWS = EPAD // 16 + 4
ROWS_W = NPAD // NS
EA_PER_W = EPAD // NW
CHUNKS_A = EA_PER_W // L
EB_PER_W = EPAD // NS
CHUNKS_B = EB_PER_W // L

f32 = jnp.float32
i32 = jnp.int32


def _mm(x, w, b, bm=1280):
    n, k = x.shape
    m = w.shape[1]

    def body(x_ref, w_ref, b_ref, o_ref):
        o_ref[...] = jnp.dot(x_ref[...], w_ref[...],
                             preferred_element_type=f32) + b_ref[...]

    return pl.pallas_call(
        body,
        grid=(n // bm,),
        in_specs=[pl.BlockSpec((bm, k), lambda i: (i, 0)),
                  pl.BlockSpec((k, m), lambda i: (0, 0)),
                  pl.BlockSpec((1, m), lambda i: (0, 0))],
        out_specs=pl.BlockSpec((bm, m), lambda i: (i, 0)),
        out_shape=jax.ShapeDtypeStruct((n, m), f32),
    )(x, w, b)


def _rden(den):
    def body(a_ref, b_ref, o_ref):
        o_ref[...] = 1.0 / (a_ref[...] + b_ref[...] + 1e-16)

    return pl.pallas_call(
        body,
        grid=(8,),
        in_specs=[pl.BlockSpec((1280, 16), lambda i: (i, 0)),
                  pl.BlockSpec((1280, 16), lambda i: (i + 8, 0))],
        out_specs=pl.BlockSpec((1280, 16), lambda i: (i, 0)),
        out_shape=jax.ShapeDtypeStruct((NPAD, 16), f32),
    )(den, den)


def _reluadd(outflat, sproj):
    def body(l_ref, r_ref, s_ref, o_ref):
        cat = jnp.concatenate([l_ref[...], r_ref[...]], axis=1)
        o_ref[...] = jnp.maximum(cat + s_ref[...], 0.0)

    return pl.pallas_call(
        body,
        grid=(8,),
        in_specs=[pl.BlockSpec((1280, 128), lambda i: (i, 0)),
                  pl.BlockSpec((1280, 128), lambda i: (i + 8, 0)),
                  pl.BlockSpec((1280, 256), lambda i: (i, 0))],
        out_specs=pl.BlockSpec((1280, 256), lambda i: (i, 0)),
        out_shape=jax.ShapeDtypeStruct((NPAD, 256), f32),
    )(outflat, outflat, sproj)


def _pool(h, bat3, w_fc, b_fc):
    def body(h_ref, b_ref, w_ref, bias_ref, o_ref, pacc, cacc):
        i = pl.program_id(0)

        @pl.when(i == 0)
        def _():
            pacc[...] = jnp.zeros_like(pacc)
            cacc[...] = jnp.zeros_like(cacc)

        bat = b_ref[0]
        gids = lax.broadcasted_iota(i32, (G, 1000), 0)
        oh = jnp.where(bat == gids, 1.0, 0.0)
        pacc[...] += jnp.dot(oh, h_ref[...], preferred_element_type=f32)
        cacc[...] += jnp.dot(oh, jnp.ones((1000, 128), f32),
                             preferred_element_type=f32)

        @pl.when(i == 9)
        def _():
            cnt = jnp.maximum(cacc[:, :1], 1.0)
            pooled = pacc[...] / cnt
            o_ref[...] = jnp.dot(pooled, w_ref[...],
                                 preferred_element_type=f32) + bias_ref[...]

    return pl.pallas_call(
        body,
        grid=(10,),
        in_specs=[pl.BlockSpec((1000, 256), lambda i: (i, 0)),
                  pl.BlockSpec((1, 1, 1000), lambda i: (i, 0, 0)),
                  pl.BlockSpec((256, 1), lambda i: (0, 0)),
                  pl.BlockSpec((1, 1), lambda i: (0, 0))],
        out_specs=pl.BlockSpec((G, 1), lambda i: (0, 0)),
        out_shape=jax.ShapeDtypeStruct((G, 1), f32),
        scratch_shapes=[pltpu.VMEM((G, 256), f32), pltpu.VMEM((G, 128), f32)],
    )(h, bat3, w_fc, b_fc)


_SC_PARAMS = dict(
    compiler_params=pltpu.CompilerParams(
        use_tc_tiling_on_sc=False, needs_layout_passes=False),
)


def _sc_phase_a(q, k, src3, dst3):
    mesh = plsc.VectorSubcoreMesh(core_axis_name="c", subcore_axis_name="s")

    @functools.partial(
        pl.kernel,
        mesh=mesh,
        out_type=(jax.ShapeDtypeStruct((EPAD + 64, 16), f32),
                  jax.ShapeDtypeStruct((2 * NPAD, 16), f32)),
        scratch_types=[
            pltpu.VMEM_SHARED((NPAD, 16), f32),
            [pltpu.VMEM((2, 16), i32)] * 2,
            [pltpu.VMEM((2, 16), i32)] * 2,
            [pltpu.VMEM((16, 256), f32)] * 4,
            [pltpu.VMEM((16, 256), f32)] * 4,
            [pltpu.VMEM((16, 16), f32)] * 2,
            [pltpu.VMEM((16,), i32)] * 2,
            pltpu.VMEM((16,), i32),
            [pltpu.VMEM((16, 16), f32)] * 4,
            [pltpu.SemaphoreType.DMA] * 4,
            [pltpu.SemaphoreType.DMA] * 4,
            [pltpu.SemaphoreType.DMA] * 2,
            [pltpu.SemaphoreType.DMA] * 2,
        ],
        **_SC_PARAMS,
    )
    def kfn(q_hbm, k_hbm, src_hbm, dst_hbm, ex_hbm, den_hbm,
            den_sh, srcslab, dstslab, qbuf, kbuf, exbuf, scidx, dumv,
            sbuf, semq, semk, semex, semsc):
        c = lax.axis_index("c")
        s = lax.axis_index("s")
        wid = c * NS + s
        wrow = wid * CHUNKS_A
        wbase = wid * EA_PER_W
        zero16 = jnp.zeros((16,), f32)
        for b in range(2):
            for r in range(16):
                exbuf[b][r, :] = zero16

        def zcopy(i, carry):
            pltpu.sync_copy(exbuf[0],
                            den_sh.at[pl.ds(s * ROWS_W + i * 16, 16)])
            return carry

        lax.fori_loop(0, ROWS_W // 16, zcopy, 0)
        plsc.subcore_barrier()

        lanes = lax.iota(i32, 16)
        dumv[...] = jnp.full((16,), N, i32)

        for sl in range(2):
            pltpu.sync_copy(src_hbm.at[pl.ds(wrow + 2 * sl, 2)], srcslab[sl])
            pltpu.sync_copy(dst_hbm.at[pl.ds(wrow + 2 * sl, 2)], dstslab[sl])
        for b in range(2):
            scidx[b][...] = dumv[...]
            pltpu.async_copy(exbuf[b], ex_hbm.at[pl.ds(EPAD + 16 * b, 16)],
                             semex[b])
            pltpu.async_copy(exbuf[b], den_sh.at[scidx[b]], semsc[b],
                             add=True)
        for sl in range(2):
            for cb in range(2):
                j = 2 * sl + cb
                pltpu.async_copy(q_hbm.at[dstslab[sl].at[cb]], qbuf[j],
                                 semq[j])
                pltpu.async_copy(k_hbm.at[srcslab[sl].at[cb]], kbuf[j],
                                 semk[j])

        def chunk(i, j, b, sl, cb):
            eb = wbase + i * L
            cur_dst = dstslab[sl].at[cb]
            cur_src = srcslab[sl].at[cb]
            pltpu.make_async_copy(q_hbm.at[cur_dst], qbuf[j], semq[j]).wait()
            pltpu.make_async_copy(k_hbm.at[cur_src], kbuf[j], semk[j]).wait()
            pltpu.make_async_copy(exbuf[b], ex_hbm.at[pl.ds(eb, L)],
                                  semex[b]).wait()
            pltpu.make_async_copy(exbuf[b], den_sh.at[scidx[b]],
                                  semsc[b]).wait()
            for e in range(16):
                for hh in range(H):
                    prods = []
                    for jj in range(4):
                        sl16 = pl.ds(hh * C + 16 * jj, 16)
                        prods.append(qbuf[j][e, sl16] * kbuf[j][e, sl16])
                    p = (prods[0] + prods[1]) + (prods[2] + prods[3])
                    sbuf[hh][e, :] = plsc.cumsum(p)
            col15 = jnp.full((16,), 15, i32)
            for hh in range(H):
                alpha = plsc.load_gather(sbuf[hh], [lanes, col15])
                exh = jnp.exp(alpha * 0.125)
                plsc.store_scatter(exbuf[b],
                                   [lanes, jnp.full((16,), hh, i32)], exh)
            scidx[b][...] = dstslab[sl][cb, :]
            pltpu.async_copy(exbuf[b], ex_hbm.at[pl.ds(eb, L)], semex[b])
            pltpu.async_copy(exbuf[b], den_sh.at[scidx[b]], semsc[b],
                             add=True)

        def quad(qi, carry):
            i0 = 4 * qi
            for pb in range(2):
                sl = pb
                for cb in range(2):
                    i = i0 + 2 * pb + cb
                    chunk(i, 2 * pb + cb, cb, sl, cb)
                rb = wrow + i0 // 2 + 2 * pb + 4
                pltpu.sync_copy(src_hbm.at[pl.ds(rb, 2)], srcslab[sl])
                pltpu.sync_copy(dst_hbm.at[pl.ds(rb, 2)], dstslab[sl])
                for cb in range(2):
                    j = 2 * pb + cb
                    pltpu.async_copy(q_hbm.at[dstslab[sl].at[cb]], qbuf[j],
                                     semq[j])
                    pltpu.async_copy(k_hbm.at[srcslab[sl].at[cb]], kbuf[j],
                                     semk[j])
            return carry

        lax.fori_loop(0, CHUNKS_A // 4, quad, 0)

        for j in range(4):
            pltpu.make_async_copy(q_hbm.at[dstslab[j // 2].at[j % 2]],
                                  qbuf[j], semq[j]).wait()
            pltpu.make_async_copy(k_hbm.at[srcslab[j // 2].at[j % 2]],
                                  kbuf[j], semk[j]).wait()
        for b in range(2):
            pltpu.make_async_copy(exbuf[b], ex_hbm.at[pl.ds(EPAD, 16)],
                                  semex[b]).wait()
            pltpu.make_async_copy(exbuf[b], den_sh.at[scidx[b]],
                                  semsc[b]).wait()
        plsc.subcore_barrier()
        pltpu.sync_copy(den_sh.at[pl.ds(s * ROWS_W, ROWS_W)],
                        den_hbm.at[pl.ds(c * NPAD + s * ROWS_W, ROWS_W)])

    return kfn(q, k, src3, dst3)


def _sc_phase_b(vstack, src3, dst3, ex, rden):
    mesh = plsc.VectorSubcoreMesh(core_axis_name="c", subcore_axis_name="s")

    @functools.partial(
        pl.kernel,
        mesh=mesh,
        out_type=jax.ShapeDtypeStruct((2 * NPAD, 128), f32),
        scratch_types=[
            pltpu.VMEM_SHARED((NPAD, 128), f32),
            [pltpu.VMEM((2, 16), i32)] * 2,
            [pltpu.VMEM((2, 16), i32)] * 2,
            [pltpu.VMEM((16, 128), f32)] * 4,
            [pltpu.VMEM((16, 128), f32)] * 2,
            [pltpu.VMEM((16, 16), f32)] * 4,
            [pltpu.VMEM((16, 16), f32)] * 4,
            pltpu.VMEM((16, 16), f32),
            [pltpu.VMEM((16,), i32)] * 4,
            [pltpu.VMEM((16,), i32)] * 2,
            pltpu.VMEM((16,), i32),
            [pltpu.SemaphoreType.DMA] * 4,
            [pltpu.SemaphoreType.DMA] * 4,
            [pltpu.SemaphoreType.DMA] * 4,
            [pltpu.SemaphoreType.DMA] * 2,
        ],
        **_SC_PARAMS,
    )
    def kfn(v_hbm, src_hbm, dst_hbm, ex_hbm, rden_hbm, out_hbm,
            oacc, srcslab, dstslab, vbuf, stage, exv, rdenv, abuf,
            sidx, scidx, dumv, semv, semr, seme, semsc):
        c = lax.axis_index("c")
        s = lax.axis_index("s")
        srow = s * CHUNKS_B
        sbase = s * EB_PER_W
        zero16 = jnp.zeros((16,), f32)
        for b in range(2):
            for r in range(16):
                for j in range(8):
                    stage[b][r, pl.ds(16 * j, 16)] = zero16

        def zcopy(i, carry):
            pltpu.sync_copy(stage[0],
                            oacc.at[pl.ds(s * ROWS_W + i * 16, 16)])
            return carry

        lax.fori_loop(0, ROWS_W // 16, zcopy, 0)
        plsc.subcore_barrier()

        lanes = lax.iota(i32, 16)
        dumv[...] = jnp.full((16,), N, i32)
        voff = c * NPAD
        h0col = c * 2

        for sl in range(2):
            pltpu.sync_copy(src_hbm.at[pl.ds(srow + 2 * sl, 2)], srcslab[sl])
            pltpu.sync_copy(dst_hbm.at[pl.ds(srow + 2 * sl, 2)], dstslab[sl])
        for b in range(2):
            scidx[b][...] = dumv[...]
            pltpu.async_copy(stage[b], oacc.at[scidx[b]], semsc[b], add=True)
        for sl in range(2):
            for cb in range(2):
                j = 2 * sl + cb
                sidx[j][...] = srcslab[sl][cb, :] + jnp.full((16,), voff,
                                                            i32)
                pltpu.async_copy(v_hbm.at[sidx[j]], vbuf[j], semv[j])
                pltpu.async_copy(rden_hbm.at[dstslab[sl].at[cb]], rdenv[j],
                                 semr[j])
                pltpu.async_copy(ex_hbm.at[pl.ds(sbase + j * L, L)], exv[j],
                                 seme[j])

        def chunk(i, j, b, sl, cb):
            eb = sbase + i * L
            cur_dst = dstslab[sl].at[cb]
            pltpu.make_async_copy(v_hbm.at[sidx[j]], vbuf[j], semv[j]).wait()
            pltpu.make_async_copy(rden_hbm.at[cur_dst], rdenv[j],
                                  semr[j]).wait()
            pltpu.make_async_copy(ex_hbm.at[pl.ds(eb, L)], exv[j],
                                  seme[j]).wait()
            pltpu.make_async_copy(stage[b], oacc.at[scidx[b]],
                                  semsc[b]).wait()
            for jj in range(2):
                colv = jnp.full((16,), h0col + jj, i32)
                aj = (plsc.load_gather(exv[j], [lanes, colv]) *
                      plsc.load_gather(rdenv[j], [lanes, colv]))
                plsc.store_scatter(abuf, [lanes, jnp.full((16,), jj, i32)],
                                   aj)
            for e in range(16):
                arow = abuf[e, :]
                a0 = arow[0]
                a1 = arow[1]
                for jj in range(8):
                    aa = a0 if jj < 4 else a1
                    stage[b][e, pl.ds(16 * jj, 16)] = (
                        vbuf[j][e, pl.ds(16 * jj, 16)] * aa)
            scidx[b][...] = dstslab[sl][cb, :]
            pltpu.async_copy(stage[b], oacc.at[scidx[b]], semsc[b], add=True)

        def quad(qi, carry):
            i0 = 4 * qi
            for pb in range(2):
                sl = pb
                for cb in range(2):
                    i = i0 + 2 * pb + cb
                    chunk(i, 2 * pb + cb, cb, sl, cb)
                rb = srow + i0 // 2 + 2 * pb + 4
                pltpu.sync_copy(src_hbm.at[pl.ds(rb, 2)], srcslab[sl])
                pltpu.sync_copy(dst_hbm.at[pl.ds(rb, 2)], dstslab[sl])
                for cb in range(2):
                    j = 2 * pb + cb
                    i4 = i0 + 2 * pb + cb + 4
                    sidx[j][...] = srcslab[sl][cb, :] + jnp.full((16,), voff,
                                                                i32)
                    pltpu.async_copy(v_hbm.at[sidx[j]], vbuf[j], semv[j])
                    pltpu.async_copy(rden_hbm.at[dstslab[sl].at[cb]],
                                     rdenv[j], semr[j])
                    pltpu.async_copy(ex_hbm.at[pl.ds(sbase + i4 * L, L)],
                                     exv[j], seme[j])
            return carry

        lax.fori_loop(0, CHUNKS_B // 4, quad, 0)

        for j in range(4):
            pltpu.make_async_copy(v_hbm.at[sidx[j]], vbuf[j], semv[j]).wait()
            pltpu.make_async_copy(rden_hbm.at[dstslab[j // 2].at[j % 2]],
                                  rdenv[j], semr[j]).wait()
            pltpu.make_async_copy(ex_hbm.at[pl.ds(sbase, L)], exv[j],
                                  seme[j]).wait()
        for b in range(2):
            pltpu.make_async_copy(stage[b], oacc.at[scidx[b]],
                                  semsc[b]).wait()
        plsc.subcore_barrier()
        pltpu.sync_copy(oacc.at[pl.ds(s * ROWS_W, ROWS_W)],
                        out_hbm.at[pl.ds(c * NPAD + s * ROWS_W, ROWS_W)])

    return kfn(vstack, src3, dst3, ex, rden)


def kernel(x, edge_index, batch, W_emb, b_emb, Wq0, bq0, Wk0, bk0, Wv0, bv0,
           Ws0, bs0, Wq1, bq1, Wk1, bk1, Wv1, bv1, Ws1, bs1, W_fc, b_fc):
    x_pad = jnp.pad(x, ((0, NPAD - N), (0, 0)))
    pad_idx = jnp.full((NROWS * 16 - E,), N, i32)
    src3 = jnp.concatenate([edge_index[0].astype(i32),
                            pad_idx]).reshape(NROWS, 16)
    dst3 = jnp.concatenate([edge_index[1].astype(i32),
                            pad_idx]).reshape(NROWS, 16)

    h = _mm(x_pad, W_emb, b_emb.reshape(1, -1))
    layers = [(Wq0, bq0, Wk0, bk0, Wv0, bv0, Ws0, bs0),
              (Wq1, bq1, Wk1, bk1, Wv1, bv1, Ws1, bs1)]
    for (Wq, bq, Wk, bk, Wv, bv, Ws, bs) in layers:
        W4 = jnp.concatenate([Wq, Wk, Wv, Ws], axis=1)
        b4 = jnp.concatenate([bq, bk, bv, bs]).reshape(1, -1)
        o = _mm(h, W4, b4)
        q = o[:, :256]
        kk = o[:, 256:512]
        vstack = jnp.concatenate([o[:, 512:640], o[:, 640:768]], axis=0)
        sproj = o[:, 768:]
        ex, den = _sc_phase_a(q, kk, src3, dst3)
        rden = _rden(den)
        outflat = _sc_phase_b(vstack, src3, dst3, ex, rden)
        h = _reluadd(outflat, sproj)

    out = _pool(h, batch.astype(i32).reshape(10, 1, 1000),
                W_fc, b_fc.reshape(1, 1))
    return out.reshape(G)
